# Initial kernel scaffold; baseline (speedup 1.0000x reference)
import functools

import jax
import jax.numpy as jnp
from jax import lax
from jax.experimental import pallas as pl
from jax.experimental.pallas import tpu as pltpu
from jax.experimental.pallas import tpu_sc as plsc

N = 10000
E = 320000
D = 128

NC = 2
NS = 16
NW = NC * NS
EPW = E // NW
CH = 80
NCHUNK = EPW // CH
RPT = N // NS
ZR = 125

_MESH = dict(core_axis_name="c", subcore_axis_name="s", num_cores=NC,
             num_subcores=NS)


@functools.partial(
    pl.kernel,
    out_type=jax.ShapeDtypeStruct((2, NW, N), jnp.float32),
    mesh=plsc.VectorSubcoreMesh(**_MESH),
    scratch_types=[
        pltpu.VMEM((NCHUNK, CH), jnp.int32),
        pltpu.VMEM((N,), jnp.float32),
        pltpu.VMEM((N,), jnp.float32),
    ],
)
def _sc_degrees(src_hbm, dst_hbm, out_hbm, idx_v, degs_v, degd_v):
    cid = lax.axis_index("c")
    sid = lax.axis_index("s")
    wid = sid * NC + cid

    zeros16 = jnp.zeros((16,), jnp.float32)
    ones16 = jnp.ones((16,), jnp.float32)

    def zero_body(i, _):
        degs_v[pl.ds(i * 16, 16)] = zeros16
        degd_v[pl.ds(i * 16, 16)] = zeros16
        return 0
    lax.fori_loop(0, N // 16, zero_body, 0)

    def count_into(deg_ref):
        def body(r, _):
            for c in range(CH // 16):
                v = idx_v[r, pl.ds(c * 16, 16)]
                plsc.addupdate_scatter(deg_ref, [v], ones16)
            return 0
        lax.fori_loop(0, NCHUNK, body, 0)

    pltpu.sync_copy(src_hbm.at[wid], idx_v)
    count_into(degs_v)
    pltpu.sync_copy(dst_hbm.at[wid], idx_v)
    count_into(degd_v)

    pltpu.sync_copy(degs_v, out_hbm.at[0, wid])
    pltpu.sync_copy(degd_v, out_hbm.at[1, wid])


@functools.partial(
    pl.kernel,
    out_type=jax.ShapeDtypeStruct((NC, N, D), jnp.float32),
    mesh=plsc.VectorSubcoreMesh(**_MESH),
    scratch_types=[
        pltpu.VMEM((NCHUNK, CH), jnp.int32),
        pltpu.VMEM((NCHUNK, CH), jnp.int32),
        pltpu.VMEM((CH, D), jnp.float32),
        pltpu.VMEM((ZR, D), jnp.float32),
        pltpu.VMEM_SHARED((N, D), jnp.float32),
        pltpu.SemaphoreType.DMA,
    ],
)
def _sc_edge(h_hbm, src_hbm, dst_hbm, out_hbm, sidx_v, didx_v, rows_v,
             zbuf_v, acc_sh, sem):
    cid = lax.axis_index("c")
    sid = lax.axis_index("s")
    wid = sid * NC + cid

    zeros16 = jnp.zeros((16,), jnp.float32)

    def zero_body(i, _):
        r = i // (D // 16)
        c = i % (D // 16)
        zbuf_v[r, pl.ds(c * 16, 16)] = zeros16
        return 0
    lax.fori_loop(0, ZR * (D // 16), zero_body, 0)
    for j in range(RPT // ZR):
        pltpu.sync_copy(zbuf_v, acc_sh.at[pl.ds(sid * RPT + j * ZR, ZR)])
    plsc.subcore_barrier()

    pltpu.sync_copy(src_hbm.at[wid], sidx_v)
    pltpu.sync_copy(dst_hbm.at[wid], didx_v)

    def chunk_body(t, _):
        pltpu.async_copy(h_hbm.at[sidx_v.at[t]], rows_v, sem).wait()
        pltpu.sync_copy(rows_v, acc_sh.at[didx_v.at[t]], add=True)
        return 0
    lax.fori_loop(0, NCHUNK, chunk_body, 0)

    plsc.subcore_barrier()
    pltpu.sync_copy(acc_sh.at[pl.ds(sid * RPT, RPT)],
                    out_hbm.at[cid, pl.ds(sid * RPT, RPT)])


BN = 1000
NB = N // BN


def _tc_pre_body(deg_ref, x_ref, w_ref, h_ref, norms_ref):
    deg = jnp.sum(deg_ref[...], axis=1)
    norms = lax.rsqrt(jnp.clip(deg, 1.0, None))
    norms_ref[...] = norms
    h = x_ref[...] * norms[0][:, None]
    h_ref[...] = jnp.dot(h, w_ref[...], preferred_element_type=jnp.float32)


def _tc_pre(deg_parts, x, w1):
    return pl.pallas_call(
        _tc_pre_body,
        grid=(NB,),
        in_specs=[
            pl.BlockSpec((2, NW, BN), lambda i: (0, 0, i)),
            pl.BlockSpec((BN, D), lambda i: (i, 0)),
            pl.BlockSpec((D, D), lambda i: (0, 0)),
        ],
        out_specs=[
            pl.BlockSpec((BN, D), lambda i: (i, 0)),
            pl.BlockSpec((2, BN), lambda i: (0, i)),
        ],
        out_shape=[
            jax.ShapeDtypeStruct((N, D), jnp.float32),
            jax.ShapeDtypeStruct((2, N), jnp.float32),
        ],
    )(deg_parts, x, w1)


def _tc_mid_body(p_ref, norms_ref, b_ref, w_ref, out_ref):
    p = p_ref[...]
    t = p[0] + p[1]
    t = t * norms_ref[1][:, None] + b_ref[...]
    t = jnp.maximum(t, 0.0) * norms_ref[0][:, None]
    out_ref[...] = jnp.dot(t, w_ref[...], preferred_element_type=jnp.float32)


def _tc_mid(parts, norms, b_prev, w_next):
    return pl.pallas_call(
        _tc_mid_body,
        grid=(NB,),
        in_specs=[
            pl.BlockSpec((NC, BN, D), lambda i: (0, i, 0)),
            pl.BlockSpec((2, BN), lambda i: (0, i)),
            pl.BlockSpec((1, D), lambda i: (0, 0)),
            pl.BlockSpec((D, D), lambda i: (0, 0)),
        ],
        out_specs=pl.BlockSpec((BN, D), lambda i: (i, 0)),
        out_shape=jax.ShapeDtypeStruct((N, D), jnp.float32),
    )(parts, norms, b_prev, w_next)


def _tc_post_body(p_ref, norms_ref, b_ref, out_ref):
    p = p_ref[...]
    t = (p[0] + p[1]) * norms_ref[1][:, None] + b_ref[...]
    out_ref[...] = jnp.maximum(t, 0.0)


def _tc_post(parts, norms, b_last):
    return pl.pallas_call(
        _tc_post_body,
        grid=(NB,),
        in_specs=[
            pl.BlockSpec((NC, BN, D), lambda i: (0, i, 0)),
            pl.BlockSpec((2, BN), lambda i: (0, i)),
            pl.BlockSpec((1, D), lambda i: (0, 0)),
        ],
        out_specs=pl.BlockSpec((BN, D), lambda i: (i, 0)),
        out_shape=jax.ShapeDtypeStruct((N, D), jnp.float32),
    )(parts, norms, b_last)


def kernel(inputs, edge_index, W1, b1, W2, b2, W3, b3):
    src3 = edge_index[0].reshape(NW, NCHUNK, CH)
    dst3 = edge_index[1].reshape(NW, NCHUNK, CH)

    deg_parts = _sc_degrees(src3, dst3)
    h, norms = _tc_pre(deg_parts, inputs, W1)

    p = _sc_edge(h, src3, dst3)
    h = _tc_mid(p, norms, b1.reshape(1, D), W2)

    p = _sc_edge(h, src3, dst3)
    h = _tc_mid(p, norms, b2.reshape(1, D), W3)

    p = _sc_edge(h, src3, dst3)
    return _tc_post(p, norms, b3.reshape(1, D))


# speedup vs baseline: 7.7323x; 7.7323x over previous
import functools

import jax
import jax.numpy as jnp
from jax import lax
from jax.experimental import pallas as pl
from jax.experimental.pallas import tpu as pltpu
from jax.experimental.pallas import tpu_sc as plsc

N = 10000
NP = 10240
E = 320000
D = 128

NC = 2
NS = 16
NW = NC * NS
EPW = E // NW
CH = 80
NCHUNK = EPW // CH
RPT = NP // NS
ZR = 128
NPB = NP // 128

_MESH = dict(core_axis_name="c", subcore_axis_name="s", num_cores=NC,
             num_subcores=NS)


@functools.partial(
    pl.kernel,
    out_type=jax.ShapeDtypeStruct((2 * NW * NP,), jnp.float32),
    mesh=plsc.VectorSubcoreMesh(**_MESH),
    compiler_params=pltpu.CompilerParams(use_tc_tiling_on_sc=False, needs_layout_passes=False),
    scratch_types=[
        pltpu.VMEM((NCHUNK, CH), jnp.int32),
        pltpu.VMEM((NP,), jnp.float32),
        pltpu.VMEM((NP,), jnp.float32),
    ],
)
def _sc_degrees(src_hbm, dst_hbm, out_hbm, idx_v, degs_v, degd_v):
    cid = lax.axis_index("c")
    sid = lax.axis_index("s")
    wid = sid * NC + cid

    zeros16 = jnp.zeros((16,), jnp.float32)
    ones16 = jnp.ones((16,), jnp.float32)

    def zero_body(i, _):
        degs_v[pl.ds(i * 16, 16)] = zeros16
        degd_v[pl.ds(i * 16, 16)] = zeros16
        return 0
    lax.fori_loop(0, NP // 16, zero_body, 0)

    def count_into(deg_ref):
        def body(r, _):
            for c in range(CH // 16):
                v = idx_v[r, pl.ds(c * 16, 16)]
                plsc.addupdate_scatter(deg_ref, [v], ones16)
            return 0
        lax.fori_loop(0, NCHUNK, body, 0)

    pltpu.sync_copy(src_hbm.at[wid], idx_v)
    count_into(degs_v)
    pltpu.sync_copy(dst_hbm.at[wid], idx_v)
    count_into(degd_v)

    pltpu.sync_copy(degs_v, out_hbm.at[pl.ds(wid * NP, NP)])
    pltpu.sync_copy(degd_v, out_hbm.at[pl.ds((NW + wid) * NP, NP)])


@functools.partial(
    pl.kernel,
    out_type=jax.ShapeDtypeStruct((NC, NP, D), jnp.float32),
    mesh=plsc.VectorSubcoreMesh(**_MESH),
    compiler_params=pltpu.CompilerParams(use_tc_tiling_on_sc=False, needs_layout_passes=False),
    scratch_types=[
        pltpu.VMEM((NCHUNK, CH), jnp.int32),
        pltpu.VMEM((NCHUNK, CH), jnp.int32),
        pltpu.VMEM((CH, D), jnp.float32),
        pltpu.VMEM((ZR, D), jnp.float32),
        pltpu.VMEM_SHARED((NP, D), jnp.float32),
        pltpu.SemaphoreType.DMA,
    ],
)
def _sc_edge(h_hbm, src_hbm, dst_hbm, out_hbm, sidx_v, didx_v, rows_v,
             zbuf_v, acc_sh, sem):
    cid = lax.axis_index("c")
    sid = lax.axis_index("s")
    wid = sid * NC + cid

    zeros16 = jnp.zeros((16,), jnp.float32)

    def zero_body(i, _):
        r = i // (D // 16)
        c = i % (D // 16)
        zbuf_v[r, pl.ds(c * 16, 16)] = zeros16
        return 0
    lax.fori_loop(0, ZR * (D // 16), zero_body, 0)
    for j in range(RPT // ZR):
        pltpu.sync_copy(zbuf_v, acc_sh.at[pl.ds(sid * RPT + j * ZR, ZR)])
    plsc.subcore_barrier()

    pltpu.sync_copy(src_hbm.at[wid], sidx_v)
    pltpu.sync_copy(dst_hbm.at[wid], didx_v)

    def chunk_body(t, _):
        pltpu.async_copy(h_hbm.at[sidx_v.at[t]], rows_v, sem).wait()
        pltpu.sync_copy(rows_v, acc_sh.at[didx_v.at[t]], add=True)
        return 0
    lax.fori_loop(0, NCHUNK, chunk_body, 0)

    plsc.subcore_barrier()
    pltpu.sync_copy(acc_sh.at[pl.ds(sid * RPT, RPT)],
                    out_hbm.at[cid, pl.ds(sid * RPT, RPT)])


BN = 1024
NB = NP // BN


def _tc_pre_body(deg_ref, x_ref, w_ref, h_ref, norms_ref):
    deg = jnp.sum(deg_ref[...], axis=1)
    norms = lax.rsqrt(jnp.clip(deg, 1.0, None))
    norms_ref[...] = norms
    h = x_ref[...] * norms[0][:, None]
    h_ref[...] = jnp.dot(h, w_ref[...], preferred_element_type=jnp.float32)


def _tc_pre(deg_parts, x, w1):
    return pl.pallas_call(
        _tc_pre_body,
        grid=(NB,),
        in_specs=[
            pl.BlockSpec((2, NW, BN), lambda i: (0, 0, i)),
            pl.BlockSpec((BN, D), lambda i: (i, 0)),
            pl.BlockSpec((D, D), lambda i: (0, 0)),
        ],
        out_specs=[
            pl.BlockSpec((BN, D), lambda i: (i, 0)),
            pl.BlockSpec((2, BN), lambda i: (0, i)),
        ],
        out_shape=[
            jax.ShapeDtypeStruct((NP, D), jnp.float32),
            jax.ShapeDtypeStruct((2, NP), jnp.float32),
        ],
    )(deg_parts, x, w1)


def _tc_mid_body(p_ref, norms_ref, b_ref, w_ref, out_ref):
    p = p_ref[...]
    t = p[0] + p[1]
    t = t * norms_ref[1][:, None] + b_ref[...]
    t = jnp.maximum(t, 0.0) * norms_ref[0][:, None]
    out_ref[...] = jnp.dot(t, w_ref[...], preferred_element_type=jnp.float32)


def _tc_mid(parts, norms, b_prev, w_next):
    return pl.pallas_call(
        _tc_mid_body,
        grid=(NB,),
        in_specs=[
            pl.BlockSpec((NC, BN, D), lambda i: (0, i, 0)),
            pl.BlockSpec((2, BN), lambda i: (0, i)),
            pl.BlockSpec((1, D), lambda i: (0, 0)),
            pl.BlockSpec((D, D), lambda i: (0, 0)),
        ],
        out_specs=pl.BlockSpec((BN, D), lambda i: (i, 0)),
        out_shape=jax.ShapeDtypeStruct((NP, D), jnp.float32),
    )(parts, norms, b_prev, w_next)


def _tc_post_body(p_ref, norms_ref, b_ref, out_ref):
    p = p_ref[...]
    t = (p[0] + p[1]) * norms_ref[1][:, None] + b_ref[...]
    out_ref[...] = jnp.maximum(t, 0.0)


def _tc_post(parts, norms, b_last):
    return pl.pallas_call(
        _tc_post_body,
        grid=(NB,),
        in_specs=[
            pl.BlockSpec((NC, BN, D), lambda i: (0, i, 0)),
            pl.BlockSpec((2, BN), lambda i: (0, i)),
            pl.BlockSpec((1, D), lambda i: (0, 0)),
        ],
        out_specs=pl.BlockSpec((BN, D), lambda i: (i, 0)),
        out_shape=jax.ShapeDtypeStruct((NP, D), jnp.float32),
    )(parts, norms, b_last)


def kernel(inputs, edge_index, W1, b1, W2, b2, W3, b3):
    src3 = edge_index[0].reshape(NW, NCHUNK, CH)
    dst3 = edge_index[1].reshape(NW, NCHUNK, CH)

    x_pad = jnp.pad(inputs, ((0, NP - N), (0, 0)))

    deg_parts = _sc_degrees(src3, dst3).reshape(2, NW, NP)
    h, norms = _tc_pre(deg_parts, x_pad, W1)

    p = _sc_edge(h, src3, dst3)
    h = _tc_mid(p, norms, b1.reshape(1, D), W2)

    p = _sc_edge(h, src3, dst3)
    h = _tc_mid(p, norms, b2.reshape(1, D), W3)

    p = _sc_edge(h, src3, dst3)
    return _tc_post(p, norms, b3.reshape(1, D))[:N]


# speedup vs baseline: 12.3716x; 1.6000x over previous
import functools

import jax
import jax.numpy as jnp
from jax import lax
from jax.experimental import pallas as pl
from jax.experimental.pallas import tpu as pltpu
from jax.experimental.pallas import tpu_sc as plsc

N = 10000
NP = 10240
E = 320000
D = 128

NC = 2
NS = 16
NW = NC * NS
EPW = E // NW
CH = 80
NCHUNK = EPW // CH
RPT = NP // NS
ZR = 40
NPB = NP // 128

_MESH = dict(core_axis_name="c", subcore_axis_name="s", num_cores=NC,
             num_subcores=NS)


@functools.partial(
    pl.kernel,
    out_type=jax.ShapeDtypeStruct((2 * NW * NP,), jnp.float32),
    mesh=plsc.VectorSubcoreMesh(**_MESH),
    compiler_params=pltpu.CompilerParams(use_tc_tiling_on_sc=False, needs_layout_passes=False),
    scratch_types=[
        pltpu.VMEM((NCHUNK, CH), jnp.int32),
        pltpu.VMEM((NP,), jnp.float32),
        pltpu.VMEM((NP,), jnp.float32),
    ],
)
def _sc_degrees(src_hbm, dst_hbm, out_hbm, idx_v, degs_v, degd_v):
    cid = lax.axis_index("c")
    sid = lax.axis_index("s")
    wid = sid * NC + cid

    zeros16 = jnp.zeros((16,), jnp.float32)
    ones16 = jnp.ones((16,), jnp.float32)

    def zero_body(i, _):
        degs_v[pl.ds(i * 16, 16)] = zeros16
        degd_v[pl.ds(i * 16, 16)] = zeros16
        return 0
    lax.fori_loop(0, NP // 16, zero_body, 0)

    def count_into(deg_ref):
        def body(r, _):
            for c in range(CH // 16):
                v = idx_v[r, pl.ds(c * 16, 16)]
                plsc.addupdate_scatter(deg_ref, [v], ones16)
            return 0
        lax.fori_loop(0, NCHUNK, body, 0)

    pltpu.sync_copy(src_hbm.at[wid], idx_v)
    count_into(degs_v)
    pltpu.sync_copy(dst_hbm.at[wid], idx_v)
    count_into(degd_v)

    pltpu.sync_copy(degs_v, out_hbm.at[pl.ds(wid * NP, NP)])
    pltpu.sync_copy(degd_v, out_hbm.at[pl.ds((NW + wid) * NP, NP)])


@functools.partial(
    pl.kernel,
    out_type=jax.ShapeDtypeStruct((NC, NP, D), jnp.float32),
    mesh=plsc.VectorSubcoreMesh(**_MESH),
    compiler_params=pltpu.CompilerParams(use_tc_tiling_on_sc=False, needs_layout_passes=False),
    scratch_types=[
        pltpu.VMEM((NCHUNK, CH), jnp.int32),
        pltpu.VMEM((NCHUNK, CH), jnp.int32),
        pltpu.VMEM((CH, D), jnp.float32),
        pltpu.VMEM((CH, D), jnp.float32),
        pltpu.VMEM((ZR, D), jnp.float32),
        pltpu.VMEM_SHARED((NP, D), jnp.float32),
        pltpu.SemaphoreType.DMA,
        pltpu.SemaphoreType.DMA,
    ],
)
def _sc_edge(h_hbm, src_hbm, dst_hbm, out_hbm, sidx_v, didx_v, rows_a,
             rows_b, zbuf_v, acc_sh, sem_a, sem_b):
    cid = lax.axis_index("c")
    sid = lax.axis_index("s")
    wid = sid * NC + cid

    zeros16 = jnp.zeros((16,), jnp.float32)

    def zero_body(i, _):
        r = i // (D // 16)
        c = i % (D // 16)
        zbuf_v[r, pl.ds(c * 16, 16)] = zeros16
        return 0
    lax.fori_loop(0, ZR * (D // 16), zero_body, 0)
    for j in range(RPT // ZR):
        pltpu.sync_copy(zbuf_v, acc_sh.at[pl.ds(sid * RPT + j * ZR, ZR)])
    plsc.subcore_barrier()

    pltpu.sync_copy(src_hbm.at[wid], sidx_v)
    pltpu.sync_copy(dst_hbm.at[wid], didx_v)

    def gather(t, buf, sem):
        pltpu.async_copy(h_hbm.at[sidx_v.at[t]], buf, sem)

    def gwait(t, buf, sem):
        pltpu.make_async_copy(h_hbm.at[sidx_v.at[t]], buf, sem).wait()

    def scatter(t, buf):
        pltpu.sync_copy(buf, acc_sh.at[didx_v.at[t]], add=True)

    gather(0, rows_a, sem_a)

    def chunk_body(i, _):
        t = 2 * i
        gather(t + 1, rows_b, sem_b)
        gwait(t, rows_a, sem_a)
        scatter(t, rows_a)
        gather(t + 2, rows_a, sem_a)
        gwait(t + 1, rows_b, sem_b)
        scatter(t + 1, rows_b)
        return 0
    lax.fori_loop(0, (NCHUNK - 1) // 2, chunk_body, 0)

    gwait(NCHUNK - 1, rows_a, sem_a)
    scatter(NCHUNK - 1, rows_a)

    plsc.subcore_barrier()
    pltpu.sync_copy(acc_sh.at[pl.ds(sid * RPT, RPT)],
                    out_hbm.at[cid, pl.ds(sid * RPT, RPT)])


BN = 1024
NB = NP // BN


def _tc_pre_body(deg_ref, x_ref, w_ref, h_ref, norms_ref):
    deg = jnp.sum(deg_ref[...], axis=1)
    norms = lax.rsqrt(jnp.clip(deg, 1.0, None))
    norms_ref[...] = norms
    h = x_ref[...] * norms[0][:, None]
    h_ref[...] = jnp.dot(h, w_ref[...], preferred_element_type=jnp.float32)


def _tc_pre(deg_parts, x, w1):
    return pl.pallas_call(
        _tc_pre_body,
        grid=(NB,),
        in_specs=[
            pl.BlockSpec((2, NW, BN), lambda i: (0, 0, i)),
            pl.BlockSpec((BN, D), lambda i: (i, 0)),
            pl.BlockSpec((D, D), lambda i: (0, 0)),
        ],
        out_specs=[
            pl.BlockSpec((BN, D), lambda i: (i, 0)),
            pl.BlockSpec((2, BN), lambda i: (0, i)),
        ],
        out_shape=[
            jax.ShapeDtypeStruct((NP, D), jnp.float32),
            jax.ShapeDtypeStruct((2, NP), jnp.float32),
        ],
    )(deg_parts, x, w1)


def _tc_mid_body(p_ref, norms_ref, b_ref, w_ref, out_ref):
    p = p_ref[...]
    t = p[0] + p[1]
    t = t * norms_ref[1][:, None] + b_ref[...]
    t = jnp.maximum(t, 0.0) * norms_ref[0][:, None]
    out_ref[...] = jnp.dot(t, w_ref[...], preferred_element_type=jnp.float32)


def _tc_mid(parts, norms, b_prev, w_next):
    return pl.pallas_call(
        _tc_mid_body,
        grid=(NB,),
        in_specs=[
            pl.BlockSpec((NC, BN, D), lambda i: (0, i, 0)),
            pl.BlockSpec((2, BN), lambda i: (0, i)),
            pl.BlockSpec((1, D), lambda i: (0, 0)),
            pl.BlockSpec((D, D), lambda i: (0, 0)),
        ],
        out_specs=pl.BlockSpec((BN, D), lambda i: (i, 0)),
        out_shape=jax.ShapeDtypeStruct((NP, D), jnp.float32),
    )(parts, norms, b_prev, w_next)


def _tc_post_body(p_ref, norms_ref, b_ref, out_ref):
    p = p_ref[...]
    t = (p[0] + p[1]) * norms_ref[1][:, None] + b_ref[...]
    out_ref[...] = jnp.maximum(t, 0.0)


def _tc_post(parts, norms, b_last):
    return pl.pallas_call(
        _tc_post_body,
        grid=(NB,),
        in_specs=[
            pl.BlockSpec((NC, BN, D), lambda i: (0, i, 0)),
            pl.BlockSpec((2, BN), lambda i: (0, i)),
            pl.BlockSpec((1, D), lambda i: (0, 0)),
        ],
        out_specs=pl.BlockSpec((BN, D), lambda i: (i, 0)),
        out_shape=jax.ShapeDtypeStruct((NP, D), jnp.float32),
    )(parts, norms, b_last)


def kernel(inputs, edge_index, W1, b1, W2, b2, W3, b3):
    src3 = edge_index[0].reshape(NW, NCHUNK, CH)
    dst3 = edge_index[1].reshape(NW, NCHUNK, CH)

    x_pad = jnp.pad(inputs, ((0, NP - N), (0, 0)))

    deg_parts = _sc_degrees(src3, dst3).reshape(2, NW, NP)
    h, norms = _tc_pre(deg_parts, x_pad, W1)

    p = _sc_edge(h, src3, dst3)
    h = _tc_mid(p, norms, b1.reshape(1, D), W2)

    p = _sc_edge(h, src3, dst3)
    h = _tc_mid(p, norms, b2.reshape(1, D), W3)

    p = _sc_edge(h, src3, dst3)
    return _tc_post(p, norms, b3.reshape(1, D))[:N]
